# Initial kernel scaffold; baseline (speedup 1.0000x reference)
import jax
import jax.numpy as jnp
from jax.experimental import pallas as pl


def kernel(x, pos, edge_index, batch, W1s, b1s, W2s, b2s, Wp1, gamma, beta, Wp2, bp2):
    raise NotImplementedError("write your pallas kernel here")


# speedup vs baseline: 1.1591x; 1.1591x over previous
import functools

import jax
import jax.numpy as jnp
from jax import lax
from jax.experimental import pallas as pl
from jax.experimental.pallas import tpu as pltpu
from jax.experimental.pallas import tpu_sc as plsc

_N = 10000
_E = 320000
_D = 128
_H = 128
_F = 8
_B = 16
_POS = 3
_FH = _F * _H
_K = 16
_NC = 2
_NS = 16
_NW = _NC * _NS
_EPW = _E // _NW
_NBLK = _EPW // _K
_S1ROWS = 2000


def _stage1_body(z_ref, w_ref, b_ref, tu_ref, tv_ref):
    acc = jnp.dot(z_ref[...], w_ref[...], preferred_element_type=jnp.float32)
    acc = acc + b_ref[...]
    tu_ref[...] = acc[:, :_FH]
    tv_ref[...] = acc[:, _FH:]


def _stage1(z, wbig, bias):
    return pl.pallas_call(
        _stage1_body,
        grid=(_N // _S1ROWS,),
        in_specs=[
            pl.BlockSpec((_S1ROWS, _D), lambda i: (i, 0)),
            pl.BlockSpec((_D, 2 * _FH), lambda i: (0, 0)),
            pl.BlockSpec((1, 2 * _FH), lambda i: (0, 0)),
        ],
        out_specs=[
            pl.BlockSpec((_S1ROWS, _FH), lambda i: (i, 0)),
            pl.BlockSpec((_S1ROWS, _FH), lambda i: (i, 0)),
        ],
        out_shape=[
            jax.ShapeDtypeStruct((_N, _FH), jnp.float32),
            jax.ShapeDtypeStruct((_N, _FH), jnp.float32),
        ],
    )(z, wbig, bias)


def _bf16_round(x):
    b = plsc.bitcast(x, jnp.int32)
    lsb = lax.bitwise_and(lax.shift_right_logical(b, 16), 1)
    r = b + jnp.int32(0x7FFF) + lsb
    r = lax.bitwise_and(r, jnp.int32(-65536))
    return plsc.bitcast(r, jnp.float32)


def _sc_body(tu, tv, posp, src, dst, batch_hbm, wpp_hbm, accp, cntp,
             srcv, dstv, buf_u, buf_v, buf_ps, buf_pd, batch_v, wpp_v,
             acc_v, cnt_v, sem_u, sem_v, sem_ps, sem_pd):
    c = lax.axis_index("c")
    s = lax.axis_index("s")
    w = c * _NS + s
    base = w * _EPW

    pltpu.sync_copy(batch_hbm, batch_v)
    pltpu.sync_copy(wpp_hbm, wpp_v)

    zv = jnp.zeros((16,), jnp.float32)
    ones = jnp.ones((16,), jnp.float32)
    zi = jnp.zeros((16,), jnp.int32)
    iota = lax.broadcasted_iota(jnp.int32, (16,), 0)

    def _zero(j, carry):
        o = j * 16
        for r in range(_B):
            acc_v[r, pl.ds(o, 16)] = zv
        return carry
    lax.fori_loop(0, _FH // 16, _zero, 0)
    for r in range(_B):
        cnt_v[r, :] = zv

    def _blk(blk, carry):
        off = base + blk * _K
        pltpu.sync_copy(src.at[pl.ds(off, _K)], srcv)
        pltpu.sync_copy(dst.at[pl.ds(off, _K)], dstv)
        cp_u = pltpu.async_copy(tu.at[dstv], buf_u, sem_u)
        cp_v = pltpu.async_copy(tv.at[srcv], buf_v, sem_v)
        cp_ps = pltpu.async_copy(posp.at[srcv], buf_ps, sem_ps)
        cp_pd = pltpu.async_copy(posp.at[dstv], buf_pd, sem_pd)
        eg = plsc.load_gather(batch_v, [dstv[...]])
        cp_ps.wait()
        cp_pd.wait()
        cp_u.wait()
        cp_v.wait()

        for half in range(2):
            rows = []
            rel0 = []
            rel1 = []
            rel2 = []
            for r8 in range(8):
                r = half * 8 + r8
                g = jnp.sum(jnp.where(iota == r, eg, 0))
                row = zi + g
                rows.append(row)
                plsc.addupdate_scatter(cnt_v, [row, iota], ones)
                relf = buf_ps[r, pl.ds(0, 16)] - buf_pd[r, pl.ds(0, 16)]
                relb = _bf16_round(relf)
                for k, lst in ((0, rel0), (1, rel1), (2, rel2)):
                    rk = jnp.sum(jnp.where(iota == k, relb, 0.0))
                    lst.append(zv + rk)

            def _ew(j, c2):
                o = j * 16
                col = o + iota
                w0 = wpp_v[0, pl.ds(o, 16)]
                w1 = wpp_v[1, pl.ds(o, 16)]
                w2 = wpp_v[2, pl.ds(o, 16)]
                for r8 in range(8):
                    r = half * 8 + r8
                    u = buf_u[r, pl.ds(o, 16)]
                    v = buf_v[r, pl.ds(o, 16)]
                    pre = (u + v) + (rel0[r8] * w0 + rel1[r8] * w1
                                     + rel2[r8] * w2)
                    h = jnp.maximum(pre, 0.0)
                    hb = _bf16_round(h)
                    plsc.addupdate_scatter(acc_v, [rows[r8], col], hb)
                return c2
            lax.fori_loop(0, _FH // 16, _ew, 0)
        return carry

    lax.fori_loop(0, _NBLK, _blk, 0)

    pltpu.sync_copy(acc_v, accp.at[w])
    pltpu.sync_copy(cnt_v, cntp.at[w])


def _stage2(tu, tv, posp, src, dst, batch, wpp):
    mesh = plsc.VectorSubcoreMesh(core_axis_name="c", subcore_axis_name="s")
    fn = functools.partial(
        pl.kernel,
        out_type=[
            jax.ShapeDtypeStruct((_NW, _B, _FH), jnp.float32),
            jax.ShapeDtypeStruct((_NW, _B, 16), jnp.float32),
        ],
        mesh=mesh,
        scratch_types=[
            pltpu.VMEM((_K,), jnp.int32),
            pltpu.VMEM((_K,), jnp.int32),
            pltpu.VMEM((_K, _FH), jnp.float32),
            pltpu.VMEM((_K, _FH), jnp.float32),
            pltpu.VMEM((_K, 128), jnp.float32),
            pltpu.VMEM((_K, 128), jnp.float32),
            pltpu.VMEM((_N,), jnp.int32),
            pltpu.VMEM((4, _FH), jnp.float32),
            pltpu.VMEM((_B, _FH), jnp.float32),
            pltpu.VMEM((_B, 16), jnp.float32),
            pltpu.SemaphoreType.DMA,
            pltpu.SemaphoreType.DMA,
            pltpu.SemaphoreType.DMA,
            pltpu.SemaphoreType.DMA,
        ],
        compiler_params=pltpu.CompilerParams(needs_layout_passes=False),
    )(_sc_body)
    return fn(tu, tv, posp, src, dst, batch, wpp)


def _head_body(accp, cntp, w2, b2, wp1, gm, bt, wp2, bp2, out):
    ph = jnp.sum(accp[...], axis=0)
    epg = jnp.sum(cntp[...], axis=0)[:, 0:1]
    ph_hi = ph.astype(jnp.bfloat16).astype(jnp.float32)
    ph_lo = (ph - ph_hi).astype(jnp.bfloat16).astype(jnp.float32)
    ph_l2 = (ph - ph_hi - ph_lo).astype(jnp.bfloat16).astype(jnp.float32)
    cols = []
    for f in range(_F):
        sl = slice(f * _H, (f + 1) * _H)
        m = (jnp.dot(ph_hi[:, sl], w2[f], preferred_element_type=jnp.float32)
             + jnp.dot(ph_lo[:, sl], w2[f], preferred_element_type=jnp.float32)
             + jnp.dot(ph_l2[:, sl], w2[f], preferred_element_type=jnp.float32))
        cols.append(m + epg * b2[f])
    pooled = jnp.concatenate(cols, axis=1)
    h = jnp.dot(pooled, wp1[...])
    mean = jnp.mean(h, axis=0, keepdims=True)
    var = jnp.mean((h - mean) ** 2, axis=0, keepdims=True)
    h = (h - mean) / jnp.sqrt(var + 1e-5) * gm[...] + bt[...]
    h = jnp.maximum(h, 0.0)
    out[...] = jnp.dot(h, wp2[...]) + bp2[...]


def _stage3(accp, cntp, w2, b2, wp1, gm, bt, wp2, bp2):
    return pl.pallas_call(
        _head_body,
        out_shape=jax.ShapeDtypeStruct((_B, 1), jnp.float32),
    )(accp, cntp, w2, b2, wp1, gm, bt, wp2, bp2)


def kernel(x, pos, edge_index, batch, W1s, b1s, W2s, b2s, Wp1, gamma, beta,
           Wp2, bp2):
    f32 = jnp.float32
    bf = lambda a: a.astype(jnp.bfloat16).astype(f32)
    src = edge_index[0]
    dst = edge_index[1]

    w_ux = jnp.transpose(W1s[:, :_D, :], (1, 0, 2)).reshape(_D, _FH)
    w_vx = jnp.transpose(W1s[:, _D:2 * _D, :], (1, 0, 2)).reshape(_D, _FH)
    w_p = jnp.transpose(W1s[:, 2 * _D:, :], (1, 0, 2)).reshape(_POS, _FH)
    wbig = jnp.concatenate([w_ux, w_vx], axis=1).astype(jnp.bfloat16)
    bias = jnp.concatenate([b1s.reshape(_FH), jnp.zeros((_FH,), f32)])
    bias = bias.reshape(1, 2 * _FH)
    z = x.astype(jnp.bfloat16)
    wpp = jnp.concatenate([bf(w_p), jnp.zeros((1, _FH), f32)], axis=0)
    posp = jnp.concatenate([pos, jnp.zeros((_N, 128 - _POS), f32)], axis=1)

    tu, tv = _stage1(z, wbig, bias)
    accp, cntp = _stage2(tu, tv, posp, src, dst, batch, wpp)
    out = _stage3(accp, cntp, bf(W2s),
                  b2s.reshape(_F, 1, _H), Wp1,
                  gamma.reshape(1, _H), beta.reshape(1, _H),
                  Wp2, bp2.reshape(1, 1))
    return out


# speedup vs baseline: 1.3527x; 1.1670x over previous
import functools

import jax
import jax.numpy as jnp
from jax import lax
from jax.experimental import pallas as pl
from jax.experimental.pallas import tpu as pltpu
from jax.experimental.pallas import tpu_sc as plsc

_N = 10000
_E = 320000
_D = 128
_H = 128
_F = 8
_B = 16
_POS = 3
_FH = _F * _H
_K = 16
_NC = 2
_NS = 16
_NW = _NC * _NS
_EPW = _E // _NW
_NBLK = _EPW // _K
_S1ROWS = 2000


def _stage1_body(z_ref, w_ref, b_ref, tu_ref, tv_ref):
    acc = jnp.dot(z_ref[...], w_ref[...], preferred_element_type=jnp.float32)
    acc = acc + b_ref[...]
    tu_ref[...] = acc[:, :_FH]
    tv_ref[...] = acc[:, _FH:]


def _stage1(z, wbig, bias):
    return pl.pallas_call(
        _stage1_body,
        grid=(_N // _S1ROWS,),
        in_specs=[
            pl.BlockSpec((_S1ROWS, _D), lambda i: (i, 0)),
            pl.BlockSpec((_D, 2 * _FH), lambda i: (0, 0)),
            pl.BlockSpec((1, 2 * _FH), lambda i: (0, 0)),
        ],
        out_specs=[
            pl.BlockSpec((_S1ROWS, _FH), lambda i: (i, 0)),
            pl.BlockSpec((_S1ROWS, _FH), lambda i: (i, 0)),
        ],
        out_shape=[
            jax.ShapeDtypeStruct((_N, _FH), jnp.float32),
            jax.ShapeDtypeStruct((_N, _FH), jnp.float32),
        ],
    )(z, wbig, bias)


def _bf16_round(x):
    b = plsc.bitcast(x, jnp.int32)
    lsb = lax.bitwise_and(lax.shift_right_logical(b, 16), 1)
    r = b + jnp.int32(0x7FFF) + lsb
    r = lax.bitwise_and(r, jnp.int32(-65536))
    return plsc.bitcast(r, jnp.float32)


def _sc_body(tu, tv, posp, src, dst, batch_hbm, wpp_hbm, accp, cntp,
             srcv0, dstv0, srcv1, dstv1, bu0, bv0, bu1, bv1,
             ps0, pd0, ps1, pd1, batch_v, wpp_v, acc_v, cnt_v,
             su0, sv0, sps0, spd0, su1, sv1, sps1, spd1):
    c = lax.axis_index("c")
    s = lax.axis_index("s")
    w = c * _NS + s
    base = w * _EPW

    pltpu.sync_copy(batch_hbm, batch_v)
    pltpu.sync_copy(wpp_hbm, wpp_v)

    zv = jnp.zeros((16,), jnp.float32)
    ones = jnp.ones((16,), jnp.float32)
    zi = jnp.zeros((16,), jnp.int32)
    iota = lax.broadcasted_iota(jnp.int32, (16,), 0)

    def _zero(j, carry):
        o = j * 16
        for r in range(_B):
            acc_v[r, pl.ds(o, 16)] = zv
        return carry
    lax.fori_loop(0, _FH // 16, _zero, 0)
    for r in range(_B):
        cnt_v[r, :] = zv

    sets = [
        (srcv0, dstv0, bu0, bv0, ps0, pd0, su0, sv0, sps0, spd0),
        (srcv1, dstv1, bu1, bv1, ps1, pd1, su1, sv1, sps1, spd1),
    ]

    def _issue(si, off):
        sv_, dv_, bu_, bv_, ps_, pd_, s1, s2, s3, s4 = sets[si]
        pltpu.sync_copy(src.at[pl.ds(off, _K)], sv_)
        pltpu.sync_copy(dst.at[pl.ds(off, _K)], dv_)
        pltpu.async_copy(tu.at[dv_], bu_, s1)
        pltpu.async_copy(tv.at[sv_], bv_, s2)
        pltpu.async_copy(posp.at[sv_], ps_, s3)
        pltpu.async_copy(posp.at[dv_], pd_, s4)

    def _wait(si):
        sv_, dv_, bu_, bv_, ps_, pd_, s1, s2, s3, s4 = sets[si]
        pltpu.make_async_copy(tu.at[dv_], bu_, s1).wait()
        pltpu.make_async_copy(tv.at[sv_], bv_, s2).wait()
        pltpu.make_async_copy(posp.at[sv_], ps_, s3).wait()
        pltpu.make_async_copy(posp.at[dv_], pd_, s4).wait()

    def _compute(si):
        sv_, dv_, bu_, bv_, ps_, pd_ = sets[si][:6]
        eg = plsc.load_gather(batch_v, [dv_[...]])
        plsc.addupdate_scatter(cnt_v, [eg, iota], ones)

        for half in range(2):
            rows = []
            rel0 = []
            rel1 = []
            rel2 = []
            for r8 in range(8):
                r = half * 8 + r8
                g = jnp.sum(jnp.where(iota == r, eg, 0))
                rows.append(zi + g)
                relf = ps_[r, pl.ds(0, 16)] - pd_[r, pl.ds(0, 16)]
                relb = _bf16_round(relf)
                for k, lst in ((0, rel0), (1, rel1), (2, rel2)):
                    rk = jnp.sum(jnp.where(iota == k, relb, 0.0))
                    lst.append(zv + rk)

            def _ew(j, c2):
                o = j * 16
                col = o + iota
                w0 = wpp_v[0, pl.ds(o, 16)]
                w1 = wpp_v[1, pl.ds(o, 16)]
                w2 = wpp_v[2, pl.ds(o, 16)]
                for r8 in range(8):
                    r = half * 8 + r8
                    u = bu_[r, pl.ds(o, 16)]
                    v = bv_[r, pl.ds(o, 16)]
                    pre = (u + v) + (rel0[r8] * w0 + rel1[r8] * w1
                                     + rel2[r8] * w2)
                    h = jnp.maximum(pre, 0.0)
                    hb = _bf16_round(h)
                    plsc.addupdate_scatter(acc_v, [rows[r8], col], hb)
                return c2
            lax.fori_loop(0, _FH // 16, _ew, 0)

    _issue(0, base)

    def _blk(i, carry):
        blk0 = 2 * i
        _issue(1, base + (blk0 + 1) * _K)
        _wait(0)
        _compute(0)
        _issue(0, base + (blk0 + 2) * _K)
        _wait(1)
        _compute(1)
        return carry

    lax.fori_loop(0, (_NBLK - 1) // 2, _blk, 0)
    _wait(0)
    _compute(0)

    pltpu.sync_copy(acc_v, accp.at[w])
    pltpu.sync_copy(cnt_v, cntp.at[w])


def _stage2(tu, tv, posp, src, dst, batch, wpp):
    mesh = plsc.VectorSubcoreMesh(core_axis_name="c", subcore_axis_name="s")
    fn = functools.partial(
        pl.kernel,
        out_type=[
            jax.ShapeDtypeStruct((_NW, _B, _FH), jnp.float32),
            jax.ShapeDtypeStruct((_NW, _B, 16), jnp.float32),
        ],
        mesh=mesh,
        scratch_types=[
            pltpu.VMEM((_K,), jnp.int32),
            pltpu.VMEM((_K,), jnp.int32),
            pltpu.VMEM((_K,), jnp.int32),
            pltpu.VMEM((_K,), jnp.int32),
            pltpu.VMEM((_K, _FH), jnp.float32),
            pltpu.VMEM((_K, _FH), jnp.float32),
            pltpu.VMEM((_K, _FH), jnp.float32),
            pltpu.VMEM((_K, _FH), jnp.float32),
            pltpu.VMEM((_K, 128), jnp.float32),
            pltpu.VMEM((_K, 128), jnp.float32),
            pltpu.VMEM((_K, 128), jnp.float32),
            pltpu.VMEM((_K, 128), jnp.float32),
            pltpu.VMEM((_N,), jnp.int32),
            pltpu.VMEM((4, _FH), jnp.float32),
            pltpu.VMEM((_B, _FH), jnp.float32),
            pltpu.VMEM((_B, 16), jnp.float32),
            pltpu.SemaphoreType.DMA,
            pltpu.SemaphoreType.DMA,
            pltpu.SemaphoreType.DMA,
            pltpu.SemaphoreType.DMA,
            pltpu.SemaphoreType.DMA,
            pltpu.SemaphoreType.DMA,
            pltpu.SemaphoreType.DMA,
            pltpu.SemaphoreType.DMA,
        ],
        compiler_params=pltpu.CompilerParams(needs_layout_passes=False),
    )(_sc_body)
    return fn(tu, tv, posp, src, dst, batch, wpp)


def _head_body(accp, cntp, w2, b2, wp1, gm, bt, wp2, bp2, out):
    ph = jnp.sum(accp[...], axis=0)
    epg = jnp.sum(cntp[...], axis=(0, 2))[:, None]
    ph_hi = ph.astype(jnp.bfloat16).astype(jnp.float32)
    ph_lo = (ph - ph_hi).astype(jnp.bfloat16).astype(jnp.float32)
    ph_l2 = (ph - ph_hi - ph_lo).astype(jnp.bfloat16).astype(jnp.float32)
    cols = []
    for f in range(_F):
        sl = slice(f * _H, (f + 1) * _H)
        m = (jnp.dot(ph_hi[:, sl], w2[f], preferred_element_type=jnp.float32)
             + jnp.dot(ph_lo[:, sl], w2[f], preferred_element_type=jnp.float32)
             + jnp.dot(ph_l2[:, sl], w2[f], preferred_element_type=jnp.float32))
        cols.append(m + epg * b2[f])
    pooled = jnp.concatenate(cols, axis=1)
    h = jnp.dot(pooled, wp1[...])
    mean = jnp.mean(h, axis=0, keepdims=True)
    var = jnp.mean((h - mean) ** 2, axis=0, keepdims=True)
    h = (h - mean) / jnp.sqrt(var + 1e-5) * gm[...] + bt[...]
    h = jnp.maximum(h, 0.0)
    out[...] = jnp.dot(h, wp2[...]) + bp2[...]


def _stage3(accp, cntp, w2, b2, wp1, gm, bt, wp2, bp2):
    return pl.pallas_call(
        _head_body,
        out_shape=jax.ShapeDtypeStruct((_B, 1), jnp.float32),
    )(accp, cntp, w2, b2, wp1, gm, bt, wp2, bp2)


def kernel(x, pos, edge_index, batch, W1s, b1s, W2s, b2s, Wp1, gamma, beta,
           Wp2, bp2):
    f32 = jnp.float32
    bf = lambda a: a.astype(jnp.bfloat16).astype(f32)
    src = edge_index[0]
    dst = edge_index[1]

    w_ux = jnp.transpose(W1s[:, :_D, :], (1, 0, 2)).reshape(_D, _FH)
    w_vx = jnp.transpose(W1s[:, _D:2 * _D, :], (1, 0, 2)).reshape(_D, _FH)
    w_p = jnp.transpose(W1s[:, 2 * _D:, :], (1, 0, 2)).reshape(_POS, _FH)
    wbig = jnp.concatenate([w_ux, w_vx], axis=1).astype(jnp.bfloat16)
    bias = jnp.concatenate([b1s.reshape(_FH), jnp.zeros((_FH,), f32)])
    bias = bias.reshape(1, 2 * _FH)
    z = x.astype(jnp.bfloat16)
    wpp = jnp.concatenate([bf(w_p), jnp.zeros((1, _FH), f32)], axis=0)
    posp = jnp.concatenate([pos, jnp.zeros((_N, 128 - _POS), f32)], axis=1)

    tu, tv = _stage1(z, wbig, bias)
    accp, cntp = _stage2(tu, tv, posp, src, dst, batch, wpp)
    out = _stage3(accp, cntp, bf(W2s),
                  b2s.reshape(_F, 1, _H), Wp1,
                  gamma.reshape(1, _H), beta.reshape(1, _H),
                  Wp2, bp2.reshape(1, 1))
    return out


# speedup vs baseline: 2.6022x; 1.9237x over previous
import functools

import jax
import jax.numpy as jnp
from jax import lax
from jax.experimental import pallas as pl
from jax.experimental.pallas import tpu as pltpu
from jax.experimental.pallas import tpu_sc as plsc

_N = 10000
_E = 320000
_D = 128
_H = 128
_F = 8
_B = 16
_POS = 3
_FH = _F * _H
_K = 16
_NC = 2
_NS = 16
_NW = _NC * _NS
_EPW = _E // _NW
_NBLK = _EPW // _K
_S1ROWS = 2000


def _stage1_body(z_ref, w_ref, b_ref, tu_ref, tv_ref):
    acc = jnp.dot(z_ref[...], w_ref[...], preferred_element_type=jnp.float32)
    acc = acc + b_ref[...]
    tu_ref[...] = acc[:, :_FH]
    tv_ref[...] = acc[:, _FH:]


def _stage1(z, wbig, bias):
    return pl.pallas_call(
        _stage1_body,
        grid=(_N // _S1ROWS,),
        in_specs=[
            pl.BlockSpec((_S1ROWS, _D), lambda i: (i, 0)),
            pl.BlockSpec((_D, 2 * _FH), lambda i: (0, 0)),
            pl.BlockSpec((1, 2 * _FH), lambda i: (0, 0)),
        ],
        out_specs=[
            pl.BlockSpec((_S1ROWS, _FH), lambda i: (i, 0)),
            pl.BlockSpec((_S1ROWS, _FH), lambda i: (i, 0)),
        ],
        out_shape=[
            jax.ShapeDtypeStruct((_N, _FH), jnp.float32),
            jax.ShapeDtypeStruct((_N, _FH), jnp.float32),
        ],
    )(z, wbig, bias)


def _bf16_round(x):
    b = plsc.bitcast(x, jnp.int32)
    lsb = lax.bitwise_and(lax.shift_right_logical(b, 16), 1)
    r = b + jnp.int32(0x7FFF) + lsb
    r = lax.bitwise_and(r, jnp.int32(-65536))
    return plsc.bitcast(r, jnp.float32)


def _sc_body(tu, tv, posp, src, dst, batch_hbm, wpp_hbm, accp, cntp,
             srcv0, dstv0, srcv1, dstv1, bu0, bv0, bu1, bv1,
             ps0, pd0, ps1, pd1, batch_v, wpp_v, acc_v, cnt_v,
             su0, sv0, sps0, spd0, su1, sv1, sps1, spd1):
    c = lax.axis_index("c")
    s = lax.axis_index("s")
    w = c * _NS + s
    base = w * _EPW

    pltpu.sync_copy(batch_hbm, batch_v)
    pltpu.sync_copy(wpp_hbm, wpp_v)

    zv = jnp.zeros((16,), jnp.float32)
    ones = jnp.ones((16,), jnp.float32)
    zi = jnp.zeros((16,), jnp.int32)
    iota = lax.broadcasted_iota(jnp.int32, (16,), 0)

    def _zero(j, carry):
        o = j * 16
        for r in range(_B):
            acc_v[r, pl.ds(o, 16)] = zv
        return carry
    lax.fori_loop(0, _FH // 16, _zero, 0)
    for r in range(_B):
        cnt_v[r, :] = zv

    sets = [
        (srcv0, dstv0, bu0, bv0, ps0, pd0, su0, sv0, sps0, spd0),
        (srcv1, dstv1, bu1, bv1, ps1, pd1, su1, sv1, sps1, spd1),
    ]

    def _issue(si, off):
        sv_, dv_, bu_, bv_, ps_, pd_, s1, s2, s3, s4 = sets[si]
        pltpu.sync_copy(src.at[pl.ds(off, _K)], sv_)
        pltpu.sync_copy(dst.at[pl.ds(off, _K)], dv_)
        pltpu.async_copy(tu.at[dv_], bu_, s1)
        pltpu.async_copy(tv.at[sv_], bv_, s2)
        pltpu.async_copy(posp.at[sv_], ps_, s3)
        pltpu.async_copy(posp.at[dv_], pd_, s4)

    def _wait(si):
        sv_, dv_, bu_, bv_, ps_, pd_, s1, s2, s3, s4 = sets[si]
        pltpu.make_async_copy(tu.at[dv_], bu_, s1).wait()
        pltpu.make_async_copy(tv.at[sv_], bv_, s2).wait()
        pltpu.make_async_copy(posp.at[sv_], ps_, s3).wait()
        pltpu.make_async_copy(posp.at[dv_], pd_, s4).wait()

    def _compute(si):
        sv_, dv_, bu_, bv_, ps_, pd_ = sets[si][:6]
        eg = plsc.load_gather(batch_v, [dv_[...]])
        plsc.addupdate_scatter(cnt_v, [eg, iota], ones)

        for half in range(2):
            rows = []
            rel0 = []
            rel1 = []
            rel2 = []
            for r8 in range(8):
                r = half * 8 + r8
                g = jnp.sum(jnp.where(iota == r, eg, 0))
                rows.append(zi + g)
                relf = ps_[r, pl.ds(0, 16)] - pd_[r, pl.ds(0, 16)]
                relb = _bf16_round(relf)
                for k, lst in ((0, rel0), (1, rel1), (2, rel2)):
                    rk = jnp.sum(jnp.where(iota == k, relb, 0.0))
                    lst.append(zv + rk)

            @plsc.parallel_loop(0, _FH, 16, unroll=2, carry=jnp.int32(0))
            def _ew(o, c2):
                col = o + iota
                w0 = wpp_v[0, pl.ds(o, 16)]
                w1 = wpp_v[1, pl.ds(o, 16)]
                w2 = wpp_v[2, pl.ds(o, 16)]
                for r8 in range(8):
                    r = half * 8 + r8
                    u = bu_[r, pl.ds(o, 16)]
                    v = bv_[r, pl.ds(o, 16)]
                    pre = (u + v) + (rel0[r8] * w0 + rel1[r8] * w1
                                     + rel2[r8] * w2)
                    h = jnp.maximum(pre, 0.0)
                    hb = _bf16_round(h)
                    plsc.addupdate_scatter(acc_v, [rows[r8], col], hb)
                return c2

    _issue(0, base)

    def _blk(i, carry):
        blk0 = 2 * i
        _issue(1, base + (blk0 + 1) * _K)
        _wait(0)
        _compute(0)
        _issue(0, base + (blk0 + 2) * _K)
        _wait(1)
        _compute(1)
        return carry

    lax.fori_loop(0, (_NBLK - 1) // 2, _blk, 0)
    _wait(0)
    _compute(0)

    pltpu.sync_copy(acc_v, accp.at[w])
    pltpu.sync_copy(cnt_v, cntp.at[w])


def _stage2(tu, tv, posp, src, dst, batch, wpp):
    mesh = plsc.VectorSubcoreMesh(core_axis_name="c", subcore_axis_name="s")
    fn = functools.partial(
        pl.kernel,
        out_type=[
            jax.ShapeDtypeStruct((_NW, _B, _FH), jnp.float32),
            jax.ShapeDtypeStruct((_NW, _B, 16), jnp.float32),
        ],
        mesh=mesh,
        scratch_types=[
            pltpu.VMEM((_K,), jnp.int32),
            pltpu.VMEM((_K,), jnp.int32),
            pltpu.VMEM((_K,), jnp.int32),
            pltpu.VMEM((_K,), jnp.int32),
            pltpu.VMEM((_K, _FH), jnp.float32),
            pltpu.VMEM((_K, _FH), jnp.float32),
            pltpu.VMEM((_K, _FH), jnp.float32),
            pltpu.VMEM((_K, _FH), jnp.float32),
            pltpu.VMEM((_K, 128), jnp.float32),
            pltpu.VMEM((_K, 128), jnp.float32),
            pltpu.VMEM((_K, 128), jnp.float32),
            pltpu.VMEM((_K, 128), jnp.float32),
            pltpu.VMEM((_N,), jnp.int32),
            pltpu.VMEM((4, _FH), jnp.float32),
            pltpu.VMEM((_B, _FH), jnp.float32),
            pltpu.VMEM((_B, 16), jnp.float32),
            pltpu.SemaphoreType.DMA,
            pltpu.SemaphoreType.DMA,
            pltpu.SemaphoreType.DMA,
            pltpu.SemaphoreType.DMA,
            pltpu.SemaphoreType.DMA,
            pltpu.SemaphoreType.DMA,
            pltpu.SemaphoreType.DMA,
            pltpu.SemaphoreType.DMA,
        ],
        compiler_params=pltpu.CompilerParams(needs_layout_passes=False),
    )(_sc_body)
    return fn(tu, tv, posp, src, dst, batch, wpp)


def _head_body(accp, cntp, w2, b2, wp1, gm, bt, wp2, bp2, out):
    ph = jnp.sum(accp[...], axis=0)
    epg = jnp.sum(cntp[...], axis=(0, 2))[:, None]
    ph_hi = ph.astype(jnp.bfloat16).astype(jnp.float32)
    ph_lo = (ph - ph_hi).astype(jnp.bfloat16).astype(jnp.float32)
    ph_l2 = (ph - ph_hi - ph_lo).astype(jnp.bfloat16).astype(jnp.float32)
    cols = []
    for f in range(_F):
        sl = slice(f * _H, (f + 1) * _H)
        m = (jnp.dot(ph_hi[:, sl], w2[f], preferred_element_type=jnp.float32)
             + jnp.dot(ph_lo[:, sl], w2[f], preferred_element_type=jnp.float32)
             + jnp.dot(ph_l2[:, sl], w2[f], preferred_element_type=jnp.float32))
        cols.append(m + epg * b2[f])
    pooled = jnp.concatenate(cols, axis=1)
    h = jnp.dot(pooled, wp1[...])
    mean = jnp.mean(h, axis=0, keepdims=True)
    var = jnp.mean((h - mean) ** 2, axis=0, keepdims=True)
    h = (h - mean) / jnp.sqrt(var + 1e-5) * gm[...] + bt[...]
    h = jnp.maximum(h, 0.0)
    out[...] = jnp.dot(h, wp2[...]) + bp2[...]


def _stage3(accp, cntp, w2, b2, wp1, gm, bt, wp2, bp2):
    return pl.pallas_call(
        _head_body,
        out_shape=jax.ShapeDtypeStruct((_B, 1), jnp.float32),
    )(accp, cntp, w2, b2, wp1, gm, bt, wp2, bp2)


def kernel(x, pos, edge_index, batch, W1s, b1s, W2s, b2s, Wp1, gamma, beta,
           Wp2, bp2):
    f32 = jnp.float32
    bf = lambda a: a.astype(jnp.bfloat16).astype(f32)
    src = edge_index[0]
    dst = edge_index[1]

    w_ux = jnp.transpose(W1s[:, :_D, :], (1, 0, 2)).reshape(_D, _FH)
    w_vx = jnp.transpose(W1s[:, _D:2 * _D, :], (1, 0, 2)).reshape(_D, _FH)
    w_p = jnp.transpose(W1s[:, 2 * _D:, :], (1, 0, 2)).reshape(_POS, _FH)
    wbig = jnp.concatenate([w_ux, w_vx], axis=1).astype(jnp.bfloat16)
    bias = jnp.concatenate([b1s.reshape(_FH), jnp.zeros((_FH,), f32)])
    bias = bias.reshape(1, 2 * _FH)
    z = x.astype(jnp.bfloat16)
    wpp = jnp.concatenate([bf(w_p), jnp.zeros((1, _FH), f32)], axis=0)
    posp = jnp.concatenate([pos, jnp.zeros((_N, 128 - _POS), f32)], axis=1)

    tu, tv = _stage1(z, wbig, bias)
    accp, cntp = _stage2(tu, tv, posp, src, dst, batch, wpp)
    out = _stage3(accp, cntp, bf(W2s),
                  b2s.reshape(_F, 1, _H), Wp1,
                  gamma.reshape(1, _H), beta.reshape(1, _H),
                  Wp2, bp2.reshape(1, 1))
    return out


# speedup vs baseline: 3.2172x; 1.2363x over previous
import functools

import jax
import jax.numpy as jnp
from jax import lax
from jax.experimental import pallas as pl
from jax.experimental.pallas import tpu as pltpu
from jax.experimental.pallas import tpu_sc as plsc

_N = 10000
_E = 320000
_D = 128
_H = 128
_F = 8
_B = 16
_POS = 3
_FH = _F * _H
_K = 16
_NC = 2
_NS = 16
_NW = _NC * _NS
_EPW = _E // _NW
_NBLK = _EPW // _K
_S1ROWS = 2000


def _stage1_body(z_ref, w_ref, b_ref, tu_ref, tv_ref):
    acc = jnp.dot(z_ref[...], w_ref[...], preferred_element_type=jnp.float32)
    acc = acc + b_ref[...]
    tu_ref[...] = acc[:, :_FH]
    tv_ref[...] = acc[:, _FH:]


def _stage1(z, wbig, bias):
    return pl.pallas_call(
        _stage1_body,
        grid=(_N // _S1ROWS,),
        in_specs=[
            pl.BlockSpec((_S1ROWS, _D), lambda i: (i, 0)),
            pl.BlockSpec((_D, 2 * _FH), lambda i: (0, 0)),
            pl.BlockSpec((1, 2 * _FH), lambda i: (0, 0)),
        ],
        out_specs=[
            pl.BlockSpec((_S1ROWS, _FH), lambda i: (i, 0)),
            pl.BlockSpec((_S1ROWS, _FH), lambda i: (i, 0)),
        ],
        out_shape=[
            jax.ShapeDtypeStruct((_N, _FH), jnp.float32),
            jax.ShapeDtypeStruct((_N, _FH), jnp.float32),
        ],
    )(z, wbig, bias)


def _bf16_round(x):
    b = plsc.bitcast(x, jnp.int32)
    lsb = lax.bitwise_and(lax.shift_right_logical(b, 16), 1)
    r = b + jnp.int32(0x7FFF) + lsb
    r = lax.bitwise_and(r, jnp.int32(-65536))
    return plsc.bitcast(r, jnp.float32)


def _sc_body(tu, tv, posp, src, dst, batch_hbm, wpp_hbm, accp, cntp,
             srcv0, dstv0, srcv1, dstv1, bu0, bv0, bu1, bv1,
             ps0, pd0, ps1, pd1, batch_v, wpp_v, acc_v, cnt_v,
             su0, sv0, sps0, spd0, su1, sv1, sps1, spd1):
    c = lax.axis_index("c")
    s = lax.axis_index("s")
    w = c * _NS + s
    base = w * _EPW

    pltpu.sync_copy(batch_hbm, batch_v)
    pltpu.sync_copy(wpp_hbm, wpp_v)

    zv = jnp.zeros((16,), jnp.float32)
    ones = jnp.ones((16,), jnp.float32)
    zi = jnp.zeros((16,), jnp.int32)
    iota = lax.broadcasted_iota(jnp.int32, (16,), 0)

    def _zero(j, carry):
        o = j * 16
        for r in range(_B):
            acc_v[r, pl.ds(o, 16)] = zv
        return carry
    lax.fori_loop(0, _FH // 16, _zero, 0)
    for r in range(_B):
        cnt_v[r, :] = zv

    sets = [
        (srcv0, dstv0, bu0, bv0, ps0, pd0, su0, sv0, sps0, spd0),
        (srcv1, dstv1, bu1, bv1, ps1, pd1, su1, sv1, sps1, spd1),
    ]

    def _issue(si, off):
        sv_, dv_, bu_, bv_, ps_, pd_, s1, s2, s3, s4 = sets[si]
        pltpu.sync_copy(src.at[pl.ds(off, _K)], sv_)
        pltpu.sync_copy(dst.at[pl.ds(off, _K)], dv_)
        pltpu.async_copy(tu.at[dv_], bu_, s1)
        pltpu.async_copy(tv.at[sv_], bv_, s2)
        pltpu.async_copy(posp.at[sv_], ps_, s3)
        pltpu.async_copy(posp.at[dv_], pd_, s4)

    def _wait(si):
        sv_, dv_, bu_, bv_, ps_, pd_, s1, s2, s3, s4 = sets[si]
        pltpu.make_async_copy(tu.at[dv_], bu_, s1).wait()
        pltpu.make_async_copy(tv.at[sv_], bv_, s2).wait()
        pltpu.make_async_copy(posp.at[sv_], ps_, s3).wait()
        pltpu.make_async_copy(posp.at[dv_], pd_, s4).wait()

    def _compute(si):
        sv_, dv_, bu_, bv_, ps_, pd_ = sets[si][:6]
        eg = plsc.load_gather(batch_v, [dv_[...]])
        plsc.addupdate_scatter(cnt_v, [eg, iota], ones)

        for half in range(2):
            rows = []
            rel0 = []
            rel1 = []
            rel2 = []
            for r8 in range(8):
                r = half * 8 + r8
                g = jnp.sum(jnp.where(iota == r, eg, 0))
                rows.append(zi + g)
                relf = ps_[r, pl.ds(0, 16)] - pd_[r, pl.ds(0, 16)]
                relb = _bf16_round(relf)
                for k, lst in ((0, rel0), (1, rel1), (2, rel2)):
                    rk = jnp.sum(jnp.where(iota == k, relb, 0.0))
                    lst.append(zv + rk)

            @plsc.parallel_loop(0, _FH, 16, unroll=4, carry=jnp.int32(0))
            def _ew(o, c2):
                col = o + iota
                w0 = wpp_v[0, pl.ds(o, 16)]
                w1 = wpp_v[1, pl.ds(o, 16)]
                w2 = wpp_v[2, pl.ds(o, 16)]
                for r8 in range(8):
                    r = half * 8 + r8
                    u = bu_[r, pl.ds(o, 16)]
                    v = bv_[r, pl.ds(o, 16)]
                    pre = (u + v) + (rel0[r8] * w0 + rel1[r8] * w1
                                     + rel2[r8] * w2)
                    h = jnp.maximum(pre, 0.0)
                    hb = _bf16_round(h)
                    plsc.addupdate_scatter(acc_v, [rows[r8], col], hb)
                return c2

    _issue(0, base)

    def _blk(i, carry):
        blk0 = 2 * i
        _issue(1, base + (blk0 + 1) * _K)
        _wait(0)
        _compute(0)
        _issue(0, base + (blk0 + 2) * _K)
        _wait(1)
        _compute(1)
        return carry

    lax.fori_loop(0, (_NBLK - 1) // 2, _blk, 0)
    _wait(0)
    _compute(0)

    pltpu.sync_copy(acc_v, accp.at[w])
    pltpu.sync_copy(cnt_v, cntp.at[w])


def _stage2(tu, tv, posp, src, dst, batch, wpp):
    mesh = plsc.VectorSubcoreMesh(core_axis_name="c", subcore_axis_name="s")
    fn = functools.partial(
        pl.kernel,
        out_type=[
            jax.ShapeDtypeStruct((_NW, _B, _FH), jnp.float32),
            jax.ShapeDtypeStruct((_NW, _B, 16), jnp.float32),
        ],
        mesh=mesh,
        scratch_types=[
            pltpu.VMEM((_K,), jnp.int32),
            pltpu.VMEM((_K,), jnp.int32),
            pltpu.VMEM((_K,), jnp.int32),
            pltpu.VMEM((_K,), jnp.int32),
            pltpu.VMEM((_K, _FH), jnp.float32),
            pltpu.VMEM((_K, _FH), jnp.float32),
            pltpu.VMEM((_K, _FH), jnp.float32),
            pltpu.VMEM((_K, _FH), jnp.float32),
            pltpu.VMEM((_K, 128), jnp.float32),
            pltpu.VMEM((_K, 128), jnp.float32),
            pltpu.VMEM((_K, 128), jnp.float32),
            pltpu.VMEM((_K, 128), jnp.float32),
            pltpu.VMEM((_N,), jnp.int32),
            pltpu.VMEM((4, _FH), jnp.float32),
            pltpu.VMEM((_B, _FH), jnp.float32),
            pltpu.VMEM((_B, 16), jnp.float32),
            pltpu.SemaphoreType.DMA,
            pltpu.SemaphoreType.DMA,
            pltpu.SemaphoreType.DMA,
            pltpu.SemaphoreType.DMA,
            pltpu.SemaphoreType.DMA,
            pltpu.SemaphoreType.DMA,
            pltpu.SemaphoreType.DMA,
            pltpu.SemaphoreType.DMA,
        ],
        compiler_params=pltpu.CompilerParams(needs_layout_passes=False),
    )(_sc_body)
    return fn(tu, tv, posp, src, dst, batch, wpp)


def _head_body(accp, cntp, w2, b2, wp1, gm, bt, wp2, bp2, out):
    ph = jnp.sum(accp[...], axis=0)
    epg = jnp.sum(cntp[...], axis=(0, 2))[:, None]
    ph_hi = ph.astype(jnp.bfloat16).astype(jnp.float32)
    ph_lo = (ph - ph_hi).astype(jnp.bfloat16).astype(jnp.float32)
    ph_l2 = (ph - ph_hi - ph_lo).astype(jnp.bfloat16).astype(jnp.float32)
    cols = []
    for f in range(_F):
        sl = slice(f * _H, (f + 1) * _H)
        m = (jnp.dot(ph_hi[:, sl], w2[f], preferred_element_type=jnp.float32)
             + jnp.dot(ph_lo[:, sl], w2[f], preferred_element_type=jnp.float32)
             + jnp.dot(ph_l2[:, sl], w2[f], preferred_element_type=jnp.float32))
        cols.append(m + epg * b2[f])
    pooled = jnp.concatenate(cols, axis=1)
    h = jnp.dot(pooled, wp1[...])
    mean = jnp.mean(h, axis=0, keepdims=True)
    var = jnp.mean((h - mean) ** 2, axis=0, keepdims=True)
    h = (h - mean) / jnp.sqrt(var + 1e-5) * gm[...] + bt[...]
    h = jnp.maximum(h, 0.0)
    out[...] = jnp.dot(h, wp2[...]) + bp2[...]


def _stage3(accp, cntp, w2, b2, wp1, gm, bt, wp2, bp2):
    return pl.pallas_call(
        _head_body,
        out_shape=jax.ShapeDtypeStruct((_B, 1), jnp.float32),
    )(accp, cntp, w2, b2, wp1, gm, bt, wp2, bp2)


def kernel(x, pos, edge_index, batch, W1s, b1s, W2s, b2s, Wp1, gamma, beta,
           Wp2, bp2):
    f32 = jnp.float32
    bf = lambda a: a.astype(jnp.bfloat16).astype(f32)
    src = edge_index[0]
    dst = edge_index[1]

    w_ux = jnp.transpose(W1s[:, :_D, :], (1, 0, 2)).reshape(_D, _FH)
    w_vx = jnp.transpose(W1s[:, _D:2 * _D, :], (1, 0, 2)).reshape(_D, _FH)
    w_p = jnp.transpose(W1s[:, 2 * _D:, :], (1, 0, 2)).reshape(_POS, _FH)
    wbig = jnp.concatenate([w_ux, w_vx], axis=1).astype(jnp.bfloat16)
    bias = jnp.concatenate([b1s.reshape(_FH), jnp.zeros((_FH,), f32)])
    bias = bias.reshape(1, 2 * _FH)
    z = x.astype(jnp.bfloat16)
    wpp = jnp.concatenate([bf(w_p), jnp.zeros((1, _FH), f32)], axis=0)
    posp = jnp.concatenate([pos, jnp.zeros((_N, 128 - _POS), f32)], axis=1)

    tu, tv = _stage1(z, wbig, bias)
    accp, cntp = _stage2(tu, tv, posp, src, dst, batch, wpp)
    out = _stage3(accp, cntp, bf(W2s),
                  b2s.reshape(_F, 1, _H), Wp1,
                  gamma.reshape(1, _H), beta.reshape(1, _H),
                  Wp2, bp2.reshape(1, 1))
    return out
